# baseline (speedup 1.0000x reference)
import functools

import jax
import jax.numpy as jnp
from jax import lax
from jax.experimental import pallas as pl
from jax.experimental.pallas import tpu as pltpu
from jax.experimental.pallas import tpu_sc as plsc

_N = 10000
_NP = 10240
_E = 160000
_EP = 163840
_P = 20000
_PP = 20480
_NC, _NS = 2, 16
_NW = _NC * _NS
_ACC_W = 48
_NCHUNK = 1


def _mesh():
    return plsc.VectorSubcoreMesh(core_axis_name="c", subcore_axis_name="s")


_SC_PARAMS = pltpu.CompilerParams(
    use_tc_tiling_on_sc=False, needs_layout_passes=False
)


def _sc_gather_rows(table, idx2d, n_rows):
    d = table.shape[1]
    per_w = n_rows // _NW
    inner = 1024 if per_w % 1024 == 0 else 512
    outer = per_w // inner
    sub = inner // 128

    nt = table.shape[0]

    @functools.partial(
        pl.kernel,
        mesh=_mesh(),
        compiler_params=_SC_PARAMS,
        out_type=jax.ShapeDtypeStruct((n_rows, d), table.dtype),
        scratch_types=[
            pltpu.VMEM((sub, 128), jnp.int32),
            pltpu.VMEM((inner, d), table.dtype),
            pltpu.VMEM_SHARED((nt, d), table.dtype),
            pltpu.SemaphoreType.DMA,
        ],
    )
    def k(table_hbm, idx_hbm, out_hbm, idx_v, rows_v, tab_sh, sem):
        sid = lax.axis_index("s")
        wid = sid * _NC + lax.axis_index("c")

        @pl.when(sid == 0)
        def _():
            pltpu.sync_copy(table_hbm, tab_sh)

        plsc.subcore_barrier()

        def body(i, carry):
            row0 = wid * (per_w // 128) + i * sub
            pltpu.sync_copy(idx_hbm.at[pl.ds(row0, sub)], idx_v)
            cps = [
                pltpu.async_copy(
                    tab_sh.at[idx_v.at[j]],
                    rows_v.at[pl.ds(j * 128, 128)],
                    sem,
                )
                for j in range(sub)
            ]
            for cp in cps:
                cp.wait()
            pltpu.sync_copy(rows_v, out_hbm.at[pl.ds(wid * per_w + i * inner, inner)])
            return carry

        lax.fori_loop(0, outer, body, 0)

    return k(table, idx2d)


def _sc_segment_acc(msg, dst2d, zeros_tab):
    n_edges = msg.shape[0]
    per_w = n_edges // _NW
    inner = 1024 if per_w % 1024 == 0 else 512
    outer = per_w // inner
    sub = inner // 128
    stripe = _NP // _NS

    @functools.partial(
        pl.kernel,
        mesh=_mesh(),
        compiler_params=_SC_PARAMS,
        out_type=jax.ShapeDtypeStruct((_NC, _NP, _ACC_W), jnp.float32),
        scratch_types=[
            pltpu.VMEM((sub, 128), jnp.int32),
            pltpu.VMEM((inner, _ACC_W), jnp.float32),
            pltpu.VMEM_SHARED((_NP, _ACC_W), jnp.float32),
            pltpu.SemaphoreType.DMA,
        ],
    )
    def k(msg_hbm, dst_hbm, zeros_hbm, out_hbm, idx_v, rows_v, acc_sh, sem):
        cid = lax.axis_index("c")
        sid = lax.axis_index("s")
        wid = sid * _NC + cid

        @pl.when(sid == 0)
        def _():
            pltpu.sync_copy(zeros_hbm, acc_sh)

        plsc.subcore_barrier()

        def body(i, carry):
            row0 = wid * (per_w // 128) + i * sub
            pltpu.sync_copy(dst_hbm.at[pl.ds(row0, sub)], idx_v)
            pltpu.sync_copy(msg_hbm.at[pl.ds(wid * per_w + i * inner, inner)], rows_v)
            cps = [
                pltpu.async_copy(
                    rows_v.at[pl.ds(j * 128, 128)],
                    acc_sh.at[idx_v.at[j]],
                    sem,
                    add=True,
                )
                for j in range(sub)
            ]
            for cp in cps:
                cp.wait()
            return carry

        lax.fori_loop(0, outer, body, 0)
        plsc.subcore_barrier()
        pltpu.sync_copy(
            acc_sh.at[pl.ds(sid * stripe, stripe)],
            out_hbm.at[cid, pl.ds(sid * stripe, stripe)],
        )

    return k(msg, dst2d, zeros_tab)


def _sc_pair_gather(eit, pipe2d, node_tab):
    per_w = _PP // _NW
    sub = per_w // 128
    ew = eit.shape[1]

    @functools.partial(
        pl.kernel,
        mesh=_mesh(),
        compiler_params=_SC_PARAMS,
        out_type=(
            jax.ShapeDtypeStruct((_PP, 32), jnp.bfloat16),
            jax.ShapeDtypeStruct((_PP, 32), jnp.bfloat16),
        ),
        scratch_types=[
            pltpu.VMEM((sub, 128), jnp.int32),
            pltpu.VMEM((per_w, ew), jnp.int32),
            pltpu.VMEM((sub, 128), jnp.int32),
            pltpu.VMEM((sub, 128), jnp.int32),
            pltpu.VMEM((per_w, 32), jnp.bfloat16),
            pltpu.VMEM((per_w, 32), jnp.bfloat16),
            pltpu.VMEM_SHARED((_NP, 32), jnp.bfloat16),
            pltpu.SemaphoreType.DMA,
        ],
    )
    def k(eit_hbm, pipe_hbm, node_hbm, pu_hbm, pv_hbm,
          pidx_v, eirows_v, srcid_v, dstid_v, urows_v, vrows_v, node_sh, sem):
        sid = lax.axis_index("s")
        wid = sid * _NC + lax.axis_index("c")
        base = wid * per_w

        @pl.when(sid == 0)
        def _():
            pltpu.sync_copy(node_hbm, node_sh)

        pltpu.sync_copy(pipe_hbm.at[pl.ds(wid * sub, sub)], pidx_v)
        cps = [
            pltpu.async_copy(
                eit_hbm.at[pidx_v.at[j]],
                eirows_v.at[pl.ds(j * 128, 128)],
                sem,
            )
            for j in range(sub)
        ]
        for cp in cps:
            cp.wait()

        col0 = jnp.zeros((16,), jnp.int32)
        col1 = jnp.ones((16,), jnp.int32)
        for j in range(per_w // 16):
            rows = lax.iota(jnp.int32, 16) + j * 16
            s = plsc.load_gather(eirows_v, [rows, col0])
            t = plsc.load_gather(eirows_v, [rows, col1])
            r, c = j // 8, (j % 8) * 16
            srcid_v[r, pl.ds(c, 16)] = s
            dstid_v[r, pl.ds(c, 16)] = t

        plsc.subcore_barrier()
        cps = [
            pltpu.async_copy(
                node_sh.at[srcid_v.at[j]],
                urows_v.at[pl.ds(j * 128, 128)],
                sem,
            )
            for j in range(sub)
        ] + [
            pltpu.async_copy(
                node_sh.at[dstid_v.at[j]],
                vrows_v.at[pl.ds(j * 128, 128)],
                sem,
            )
            for j in range(sub)
        ]
        for cp in cps:
            cp.wait()
        pltpu.sync_copy(urows_v, pu_hbm.at[pl.ds(base, per_w)])
        pltpu.sync_copy(vrows_v, pv_hbm.at[pl.ds(base, per_w)])

    return k(eit, pipe2d, node_tab)


_T_MSG = 2000


def _msg_body(nin):
    def body(ea_ref, xj_ref, w1_ref, b1_ref, w2_ref, b2_ref, rep_ref,
             fold_ref, cnt_ref, out_ref):
        h = jnp.maximum(
            jnp.dot(ea_ref[:], w1_ref[:], preferred_element_type=jnp.float32)
            + b1_ref[:],
            0.0,
        )
        wt = (
            jnp.dot(h.astype(jnp.bfloat16), w2_ref[:],
                    preferred_element_type=jnp.float32)
            + b2_ref[:]
        ).astype(jnp.bfloat16)
        xr = jnp.dot(xj_ref[:], rep_ref[:],
                     preferred_element_type=jnp.float32).astype(jnp.bfloat16)
        m = wt * xr
        out_ref[:] = (
            jnp.dot(m, fold_ref[:], preferred_element_type=jnp.float32)
            + cnt_ref[:]
        )

    return body


def _tc_msg(ea, xj, w1, b1r, w2b, b2r, rep, fold, cntb, nin):
    ne = ea.shape[0]
    grid = ne // _T_MSG
    kin = w1.shape[1]
    kw = w2b.shape[0]
    return pl.pallas_call(
        _msg_body(nin),
        grid=(grid,),
        in_specs=[
            pl.BlockSpec((_T_MSG, 16), lambda i: (i, 0)),
            pl.BlockSpec((_T_MSG, nin), lambda i: (i, 0)),
            pl.BlockSpec((16, kin), lambda i: (0, 0)),
            pl.BlockSpec((1, kin), lambda i: (0, 0)),
            pl.BlockSpec((kw, nin * 32), lambda i: (0, 0)),
            pl.BlockSpec((1, nin * 32), lambda i: (0, 0)),
            pl.BlockSpec((nin, nin * 32), lambda i: (0, 0)),
            pl.BlockSpec((nin * 32, _ACC_W), lambda i: (0, 0)),
            pl.BlockSpec((1, _ACC_W), lambda i: (0, 0)),
        ],
        out_specs=pl.BlockSpec((_T_MSG, _ACC_W), lambda i: (i, 0)),
        out_shape=jax.ShapeDtypeStruct((_EP, _ACC_W), jnp.float32),
    )(ea, xj, w1, b1r, w2b, b2r, rep, fold, cntb)


def _node_body(*refs):
    nparts = len(refs) - 6
    part_refs = refs[:nparts]
    x_ref, root_ref, bias_ref, g_ref, b_ref, out_ref = refs[nparts:]
    acc = part_refs[0][0] + part_refs[0][1]
    for pr in part_refs[1:]:
        acc = acc + pr[0] + pr[1]
    ssum = acc[:, 0:32]
    cnt = acc[:, 32:33]
    mean = ssum / jnp.maximum(cnt, 1.0)
    h = (
        mean
        + jnp.dot(x_ref[:], root_ref[:], preferred_element_type=jnp.float32)
        + bias_ref[:]
    )
    r = jnp.maximum(h, 0.0)
    mu = jnp.mean(r, axis=1, keepdims=True)
    var = jnp.mean((r - mu) ** 2, axis=1, keepdims=True)
    out_ref[:] = ((r - mu) * lax.rsqrt(var + 1e-5) * g_ref[:]
                  + b_ref[:]).astype(out_ref.dtype)


def _tc_node(parts, xin, root, biasr, lgr, lbr):
    din = xin.shape[1]
    part_specs = [pl.BlockSpec((_NC, _NP, _ACC_W), lambda: (0, 0, 0))
                  for _ in parts]
    return pl.pallas_call(
        _node_body,
        in_specs=part_specs + [
            pl.BlockSpec((_NP, din), lambda: (0, 0)),
            pl.BlockSpec((din, 32), lambda: (0, 0)),
            pl.BlockSpec((1, 32), lambda: (0, 0)),
            pl.BlockSpec((1, 32), lambda: (0, 0)),
            pl.BlockSpec((1, 32), lambda: (0, 0)),
        ],
        out_specs=pl.BlockSpec((_NP, 32), lambda: (0, 0)),
        out_shape=jax.ShapeDtypeStruct((_NP, 32), jnp.bfloat16),
    )(*parts, xin, root, biasr, lgr, lbr)


def _node2_body(*refs):
    nparts = len(refs) - 7
    part_refs = refs[:nparts]
    x_ref, root_ref, bias_ref, g_ref, b_ref, out_ref, gvec_ref = refs[nparts:]
    acc = part_refs[0][0] + part_refs[0][1]
    for pr in part_refs[1:]:
        acc = acc + pr[0] + pr[1]
    ssum = acc[:, 0:32]
    cnt = acc[:, 32:33]
    mean = ssum / jnp.maximum(cnt, 1.0)
    h = (
        mean
        + jnp.dot(x_ref[:], root_ref[:], preferred_element_type=jnp.float32)
        + bias_ref[:]
    )
    r = jnp.maximum(h, 0.0)
    mu = jnp.mean(r, axis=1, keepdims=True)
    var = jnp.mean((r - mu) ** 2, axis=1, keepdims=True)
    emb = (r - mu) * lax.rsqrt(var + 1e-5) * g_ref[:] + b_ref[:]
    out_ref[:] = emb.astype(out_ref.dtype)
    gvec_ref[:] = jnp.sum(emb[0:_N, :], axis=0, keepdims=True) * (1.0 / _N)


def _tc_node2(parts, xin, root, biasr, lgr, lbr):
    din = xin.shape[1]
    part_specs = [pl.BlockSpec((_NC, _NP, _ACC_W), lambda: (0, 0, 0))
                  for _ in parts]
    return pl.pallas_call(
        _node2_body,
        in_specs=part_specs + [
            pl.BlockSpec((_NP, din), lambda: (0, 0)),
            pl.BlockSpec((din, 32), lambda: (0, 0)),
            pl.BlockSpec((1, 32), lambda: (0, 0)),
            pl.BlockSpec((1, 32), lambda: (0, 0)),
            pl.BlockSpec((1, 32), lambda: (0, 0)),
        ],
        out_specs=[
            pl.BlockSpec((_NP, 32), lambda: (0, 0)),
            pl.BlockSpec((1, 32), lambda: (0, 0)),
        ],
        out_shape=[
            jax.ShapeDtypeStruct((_NP, 32), jnp.bfloat16),
            jax.ShapeDtypeStruct((1, 32), jnp.float32),
        ],
    )(*parts, xin, root, biasr, lgr, lbr)


def _head_body(pu_ref, pv_ref, pwu_ref, pwv_ref, pb_ref, plg_ref, plb_ref,
               qw_ref, qb_ref, gvec_ref, ghw_ref, ghb_ref, ghg_ref, ghbb_ref,
               qgw_ref, qgb_ref, mask_ref, out_ref):
    pre = (
        jnp.dot(pu_ref[:], pwu_ref[:], preferred_element_type=jnp.float32)
        + jnp.dot(pv_ref[:], pwv_ref[:], preferred_element_type=jnp.float32)
        + pb_ref[:]
    )
    r = jnp.maximum(pre, 0.0)
    mu = jnp.mean(r, axis=1, keepdims=True)
    var = jnp.mean((r - mu) ** 2, axis=1, keepdims=True)
    npf = (r - mu) * lax.rsqrt(var + 1e-5) * plg_ref[:] + plb_ref[:]
    qn = jnp.dot(npf, qw_ref[:], preferred_element_type=jnp.float32) + qb_ref[:]

    g = gvec_ref[:]
    gr = jnp.maximum(
        jnp.dot(g, ghw_ref[:], preferred_element_type=jnp.float32) + ghb_ref[:],
        0.0,
    )
    gmu = jnp.mean(gr, axis=1, keepdims=True)
    gvar = jnp.mean((gr - gmu) ** 2, axis=1, keepdims=True)
    gh = (gr - gmu) * lax.rsqrt(gvar + 1e-5) * ghg_ref[:] + ghbb_ref[:]
    qg = jnp.dot(gh, qgw_ref[:], preferred_element_type=jnp.float32) + qgb_ref[:]

    q = qn + qg
    m = mask_ref[:]
    neg = jnp.float32(-1e9)
    o0 = jnp.where(m < 0.5, neg, q[:, 0:1])
    o1 = jnp.where((1.0 - m) < 0.5, neg, q[:, 1:2])
    out_ref[:] = jnp.concatenate([o0, o1], axis=1)


_T_HEAD = 2048


def _tc_head(pu, pv, pwu, pwv, pbr, plgr, plbr, qw, qbr,
             gvec, ghw, ghbr, ghgr, ghbbr, qgw, qgbr, maskc):
    row = lambda w: pl.BlockSpec((_T_HEAD, w), lambda i: (i, 0))
    full = lambda s: pl.BlockSpec(s, lambda i: tuple(0 for _ in s))
    return pl.pallas_call(
        _head_body,
        grid=(_PP // _T_HEAD,),
        in_specs=[
            row(32), row(32),
            full((32, 32)), full((32, 32)), full((1, 32)),
            full((1, 32)), full((1, 32)),
            full((32, 2)), full((1, 2)),
            full((1, 32)), full((32, 32)), full((1, 32)),
            full((1, 32)), full((1, 32)),
            full((32, 2)), full((1, 2)),
            row(1),
        ],
        out_specs=row(2),
        out_shape=jax.ShapeDtypeStruct((_PP, 2), jnp.float32),
    )(pu, pv, pwu, pwv, pbr, plgr, plbr, qw, qbr,
      gvec, ghw, ghbr, ghgr, ghbbr, qgw, qgbr, maskc)


def kernel(x, edge_index, edge_attr, pipe_edge_idx, pipe_open_mask, batch, params):
    p = params
    f32 = jnp.float32
    i32 = jnp.int32

    src = edge_index[0]
    dst = edge_index[1]
    epad = _EP - _E
    srcp = jnp.concatenate([src, jnp.zeros((epad,), i32)]).reshape(_EP // 128, 128)
    dstp = jnp.concatenate([dst, jnp.full((epad,), _N, i32)]).reshape(_EP // 128, 128)
    xpad = jnp.pad(x, ((0, _NP - _N), (0, 0))).astype(jnp.bfloat16)
    zeros_tab = jnp.zeros((_NP, _ACC_W), f32)
    eit = jnp.pad(edge_index.T, ((0, 0), (0, 6)))
    pipe2d = jnp.pad(pipe_edge_idx, (0, _PP - _P)).reshape(_PP // 128, 128)
    maskc = jnp.pad(pipe_open_mask, (0, _PP - _P)).reshape(_PP, 1)

    bf16 = jnp.bfloat16
    w1a, b1a = p["enn1_w1"], p["enn1_b1"].reshape(1, -1)
    w2a, b2a = p["enn1_w2"].astype(bf16), p["enn1_b2"].reshape(1, -1)
    w1b, b1b = p["enn2_w1"], p["enn2_b1"].reshape(1, -1)
    w2b, b2b = p["enn2_w2"].astype(bf16), p["enn2_b2"].reshape(1, -1)

    def rep_mat(nin):
        nw = nin * 32
        return (jnp.arange(nw)[None, :] // 32
                == jnp.arange(nin)[:, None]).astype(bf16)

    rep64 = rep_mat(64)
    rep32 = rep_mat(32)

    def fold_mat(nin):
        nw = nin * 32
        return (jnp.arange(nw)[:, None] % 32
                == jnp.arange(_ACC_W)[None, :]).astype(bf16)

    fold64 = fold_mat(64)
    fold32 = fold_mat(32)
    cntb = jnp.zeros((1, _ACC_W), f32).at[0, 32].set(1.0)

    xj = _sc_gather_rows(xpad, srcp, _EP)
    msg1 = _tc_msg(edge_attr, xj, w1a, b1a, w2a, b2a, rep64, fold64, cntb, 64)
    parts1 = [_sc_segment_acc(msg1, dstp, zeros_tab)]
    h1 = _tc_node(parts1, xpad, p["root1"].astype(bf16), p["bias1"].reshape(1, -1),
                  p["ln1_g"].reshape(1, -1), p["ln1_b"].reshape(1, -1))

    hj = _sc_gather_rows(h1, srcp, _EP)
    msg2 = _tc_msg(edge_attr, hj, w1b, b1b, w2b, b2b, rep32, fold32, cntb, 32)
    parts2 = [_sc_segment_acc(msg2, dstp, zeros_tab)]
    node_emb, gvec = _tc_node2(parts2, h1, p["root2"].astype(bf16), p["bias2"].reshape(1, -1),
                               p["ln2_g"].reshape(1, -1), p["ln2_b"].reshape(1, -1))

    pu, pv = _sc_pair_gather(eit, pipe2d, node_emb)
    qs = _tc_head(
        pu, pv,
        p["pair_w"][:32].astype(bf16), p["pair_w"][32:].astype(bf16),
        p["pair_b"].reshape(1, -1),
        p["pair_ln_g"].reshape(1, -1), p["pair_ln_b"].reshape(1, -1),
        p["qnode_w"], p["qnode_b"].reshape(1, -1),
        gvec, p["gh_w"], p["gh_b"].reshape(1, -1),
        p["gh_ln_g"].reshape(1, -1), p["gh_ln_b"].reshape(1, -1),
        p["qg_w"], p["qg_b"].reshape(1, -1),
        maskc,
    )
    return qs[:_P].reshape(1, -1)


# speedup vs baseline: 1.3308x; 1.3308x over previous
import functools

import jax
import jax.numpy as jnp
from jax import lax
from jax.experimental import pallas as pl
from jax.experimental.pallas import tpu as pltpu
from jax.experimental.pallas import tpu_sc as plsc

_N = 10000
_NP = 10240
_E = 160000
_EP = 163840
_P = 20000
_PP = 20480
_NC, _NS = 2, 16
_NW = _NC * _NS
_ACC_W = 48
_NCHUNK = 1


def _mesh():
    return plsc.VectorSubcoreMesh(core_axis_name="c", subcore_axis_name="s")


_SC_PARAMS = pltpu.CompilerParams(
    use_tc_tiling_on_sc=False, needs_layout_passes=False
)


def _sc_gather_rows(table, idx2d, n_rows):
    d = table.shape[1]
    per_w = n_rows // _NW
    inner = 1024 if per_w % 1024 == 0 else 512
    outer = per_w // inner
    sub = inner // 128

    nt = table.shape[0]

    @functools.partial(
        pl.kernel,
        mesh=_mesh(),
        compiler_params=_SC_PARAMS,
        out_type=jax.ShapeDtypeStruct((n_rows, d), table.dtype),
        scratch_types=[
            pltpu.VMEM((sub, 128), jnp.int32),
            pltpu.VMEM((inner, d), table.dtype),
            pltpu.VMEM_SHARED((nt, d), table.dtype),
            pltpu.SemaphoreType.DMA,
        ],
    )
    def k(table_hbm, idx_hbm, out_hbm, idx_v, rows_v, tab_sh, sem):
        sid = lax.axis_index("s")
        wid = sid * _NC + lax.axis_index("c")

        @pl.when(sid == 0)
        def _():
            pltpu.sync_copy(table_hbm, tab_sh)

        plsc.subcore_barrier()

        def body(i, carry):
            row0 = wid * (per_w // 128) + i * sub
            pltpu.sync_copy(idx_hbm.at[pl.ds(row0, sub)], idx_v)
            cps = [
                pltpu.async_copy(
                    tab_sh.at[idx_v.at[j]],
                    rows_v.at[pl.ds(j * 128, 128)],
                    sem,
                )
                for j in range(sub)
            ]
            for cp in cps:
                cp.wait()
            pltpu.sync_copy(rows_v, out_hbm.at[pl.ds(wid * per_w + i * inner, inner)])
            return carry

        lax.fori_loop(0, outer, body, 0)

    return k(table, idx2d)


def _sc_segment_acc(msg, dst2d, zeros_tab):
    n_edges = msg.shape[0]
    per_w = n_edges // _NW
    inner = 1024 if per_w % 1024 == 0 else 512
    outer = per_w // inner
    sub = inner // 128
    stripe = _NP // _NS

    @functools.partial(
        pl.kernel,
        mesh=_mesh(),
        compiler_params=_SC_PARAMS,
        out_type=jax.ShapeDtypeStruct((_NC, _NP, _ACC_W), jnp.float32),
        scratch_types=[
            pltpu.VMEM((sub, 128), jnp.int32),
            pltpu.VMEM((inner, _ACC_W), jnp.float32),
            pltpu.VMEM_SHARED((_NP, _ACC_W), jnp.float32),
            pltpu.SemaphoreType.DMA,
        ],
    )
    def k(msg_hbm, dst_hbm, zeros_hbm, out_hbm, idx_v, rows_v, acc_sh, sem):
        cid = lax.axis_index("c")
        sid = lax.axis_index("s")
        wid = sid * _NC + cid

        @pl.when(sid == 0)
        def _():
            pltpu.sync_copy(zeros_hbm, acc_sh)

        plsc.subcore_barrier()

        def body(i, carry):
            row0 = wid * (per_w // 128) + i * sub
            pltpu.sync_copy(dst_hbm.at[pl.ds(row0, sub)], idx_v)
            pltpu.sync_copy(msg_hbm.at[pl.ds(wid * per_w + i * inner, inner)], rows_v)
            cps = [
                pltpu.async_copy(
                    rows_v.at[pl.ds(j * 128, 128)],
                    acc_sh.at[idx_v.at[j]],
                    sem,
                    add=True,
                )
                for j in range(sub)
            ]
            for cp in cps:
                cp.wait()
            return carry

        lax.fori_loop(0, outer, body, 0)
        plsc.subcore_barrier()
        pltpu.sync_copy(
            acc_sh.at[pl.ds(sid * stripe, stripe)],
            out_hbm.at[cid, pl.ds(sid * stripe, stripe)],
        )

    return k(msg, dst2d, zeros_tab)


def _sc_pair_gather(eit, pipe2d, node_tab):
    per_w = _PP // _NW
    sub = per_w // 128
    ew = eit.shape[1]

    @functools.partial(
        pl.kernel,
        mesh=_mesh(),
        compiler_params=_SC_PARAMS,
        out_type=(
            jax.ShapeDtypeStruct((_PP, 32), jnp.bfloat16),
            jax.ShapeDtypeStruct((_PP, 32), jnp.bfloat16),
        ),
        scratch_types=[
            pltpu.VMEM((sub, 128), jnp.int32),
            pltpu.VMEM((per_w, ew), jnp.int32),
            pltpu.VMEM((sub, 128), jnp.int32),
            pltpu.VMEM((sub, 128), jnp.int32),
            pltpu.VMEM((per_w, 32), jnp.bfloat16),
            pltpu.VMEM((per_w, 32), jnp.bfloat16),
            pltpu.VMEM_SHARED((_NP, 32), jnp.bfloat16),
            pltpu.SemaphoreType.DMA,
        ],
    )
    def k(eit_hbm, pipe_hbm, node_hbm, pu_hbm, pv_hbm,
          pidx_v, eirows_v, srcid_v, dstid_v, urows_v, vrows_v, node_sh, sem):
        sid = lax.axis_index("s")
        wid = sid * _NC + lax.axis_index("c")
        base = wid * per_w

        @pl.when(sid == 0)
        def _():
            pltpu.sync_copy(node_hbm, node_sh)

        pltpu.sync_copy(pipe_hbm.at[pl.ds(wid * sub, sub)], pidx_v)
        cps = [
            pltpu.async_copy(
                eit_hbm.at[pidx_v.at[j]],
                eirows_v.at[pl.ds(j * 128, 128)],
                sem,
            )
            for j in range(sub)
        ]
        for cp in cps:
            cp.wait()

        col0 = jnp.zeros((16,), jnp.int32)
        col1 = jnp.ones((16,), jnp.int32)
        for j in range(per_w // 16):
            rows = lax.iota(jnp.int32, 16) + j * 16
            s = plsc.load_gather(eirows_v, [rows, col0])
            t = plsc.load_gather(eirows_v, [rows, col1])
            r, c = j // 8, (j % 8) * 16
            srcid_v[r, pl.ds(c, 16)] = s
            dstid_v[r, pl.ds(c, 16)] = t

        plsc.subcore_barrier()
        cps = [
            pltpu.async_copy(
                node_sh.at[srcid_v.at[j]],
                urows_v.at[pl.ds(j * 128, 128)],
                sem,
            )
            for j in range(sub)
        ] + [
            pltpu.async_copy(
                node_sh.at[dstid_v.at[j]],
                vrows_v.at[pl.ds(j * 128, 128)],
                sem,
            )
            for j in range(sub)
        ]
        for cp in cps:
            cp.wait()
        pltpu.sync_copy(urows_v, pu_hbm.at[pl.ds(base, per_w)])
        pltpu.sync_copy(vrows_v, pv_hbm.at[pl.ds(base, per_w)])

    return k(eit, pipe2d, node_tab)


_T_MSG = 4000


def _msg_body(nin):
    nw = nin * 32

    def body(ea_ref, xj_ref, w1_ref, b1_ref, w2_ref, b2_ref, rep_ref, out_ref):
        h = jnp.maximum(
            jnp.dot(ea_ref[:], w1_ref[:], preferred_element_type=jnp.float32)
            + b1_ref[:],
            0.0,
        )
        wt = (
            jnp.dot(h.astype(jnp.bfloat16), w2_ref[:],
                    preferred_element_type=jnp.float32)
            + b2_ref[:]
        ).astype(jnp.bfloat16)
        xr = jnp.dot(xj_ref[:], rep_ref[:],
                     preferred_element_type=jnp.float32).astype(jnp.bfloat16)
        m = wt * xr
        w = nw
        while w > 64:
            w //= 2
            m = m[:, :w] + m[:, w:]
        m = m[:, :32].astype(jnp.float32) + m[:, 32:].astype(jnp.float32)
        t = m.shape[0]
        out_ref[:] = jnp.concatenate(
            [m, jnp.ones((t, 1), jnp.float32), jnp.zeros((t, 15), jnp.float32)],
            axis=1,
        )

    return body


def _tc_msg(ea, xj, w1, b1r, w2b, b2r, rep, nin):
    ne = ea.shape[0]
    grid = ne // _T_MSG
    kin = w1.shape[1]
    kw = w2b.shape[0]
    return pl.pallas_call(
        _msg_body(nin),
        grid=(grid,),
        in_specs=[
            pl.BlockSpec((_T_MSG, 16), lambda i: (i, 0)),
            pl.BlockSpec((_T_MSG, nin), lambda i: (i, 0)),
            pl.BlockSpec((16, kin), lambda i: (0, 0)),
            pl.BlockSpec((1, kin), lambda i: (0, 0)),
            pl.BlockSpec((kw, nin * 32), lambda i: (0, 0)),
            pl.BlockSpec((1, nin * 32), lambda i: (0, 0)),
            pl.BlockSpec((nin, nin * 32), lambda i: (0, 0)),
        ],
        out_specs=pl.BlockSpec((_T_MSG, _ACC_W), lambda i: (i, 0)),
        out_shape=jax.ShapeDtypeStruct((_EP, _ACC_W), jnp.float32),
    )(ea, xj, w1, b1r, w2b, b2r, rep)


def _node_body(*refs):
    nparts = len(refs) - 6
    part_refs = refs[:nparts]
    x_ref, root_ref, bias_ref, g_ref, b_ref, out_ref = refs[nparts:]
    acc = part_refs[0][0] + part_refs[0][1]
    for pr in part_refs[1:]:
        acc = acc + pr[0] + pr[1]
    ssum = acc[:, 0:32]
    cnt = acc[:, 32:33]
    mean = ssum / jnp.maximum(cnt, 1.0)
    h = (
        mean
        + jnp.dot(x_ref[:], root_ref[:], preferred_element_type=jnp.float32)
        + bias_ref[:]
    )
    r = jnp.maximum(h, 0.0)
    mu = jnp.mean(r, axis=1, keepdims=True)
    var = jnp.mean((r - mu) ** 2, axis=1, keepdims=True)
    out_ref[:] = ((r - mu) * lax.rsqrt(var + 1e-5) * g_ref[:]
                  + b_ref[:]).astype(out_ref.dtype)


def _tc_node(parts, xin, root, biasr, lgr, lbr):
    din = xin.shape[1]
    part_specs = [pl.BlockSpec((_NC, _NP, _ACC_W), lambda: (0, 0, 0))
                  for _ in parts]
    return pl.pallas_call(
        _node_body,
        in_specs=part_specs + [
            pl.BlockSpec((_NP, din), lambda: (0, 0)),
            pl.BlockSpec((din, 32), lambda: (0, 0)),
            pl.BlockSpec((1, 32), lambda: (0, 0)),
            pl.BlockSpec((1, 32), lambda: (0, 0)),
            pl.BlockSpec((1, 32), lambda: (0, 0)),
        ],
        out_specs=pl.BlockSpec((_NP, 32), lambda: (0, 0)),
        out_shape=jax.ShapeDtypeStruct((_NP, 32), jnp.bfloat16),
    )(*parts, xin, root, biasr, lgr, lbr)


def _node2_body(*refs):
    nparts = len(refs) - 7
    part_refs = refs[:nparts]
    x_ref, root_ref, bias_ref, g_ref, b_ref, out_ref, gvec_ref = refs[nparts:]
    acc = part_refs[0][0] + part_refs[0][1]
    for pr in part_refs[1:]:
        acc = acc + pr[0] + pr[1]
    ssum = acc[:, 0:32]
    cnt = acc[:, 32:33]
    mean = ssum / jnp.maximum(cnt, 1.0)
    h = (
        mean
        + jnp.dot(x_ref[:], root_ref[:], preferred_element_type=jnp.float32)
        + bias_ref[:]
    )
    r = jnp.maximum(h, 0.0)
    mu = jnp.mean(r, axis=1, keepdims=True)
    var = jnp.mean((r - mu) ** 2, axis=1, keepdims=True)
    emb = (r - mu) * lax.rsqrt(var + 1e-5) * g_ref[:] + b_ref[:]
    out_ref[:] = emb.astype(out_ref.dtype)
    gvec_ref[:] = jnp.sum(emb[0:_N, :], axis=0, keepdims=True) * (1.0 / _N)


def _tc_node2(parts, xin, root, biasr, lgr, lbr):
    din = xin.shape[1]
    part_specs = [pl.BlockSpec((_NC, _NP, _ACC_W), lambda: (0, 0, 0))
                  for _ in parts]
    return pl.pallas_call(
        _node2_body,
        in_specs=part_specs + [
            pl.BlockSpec((_NP, din), lambda: (0, 0)),
            pl.BlockSpec((din, 32), lambda: (0, 0)),
            pl.BlockSpec((1, 32), lambda: (0, 0)),
            pl.BlockSpec((1, 32), lambda: (0, 0)),
            pl.BlockSpec((1, 32), lambda: (0, 0)),
        ],
        out_specs=[
            pl.BlockSpec((_NP, 32), lambda: (0, 0)),
            pl.BlockSpec((1, 32), lambda: (0, 0)),
        ],
        out_shape=[
            jax.ShapeDtypeStruct((_NP, 32), jnp.bfloat16),
            jax.ShapeDtypeStruct((1, 32), jnp.float32),
        ],
    )(*parts, xin, root, biasr, lgr, lbr)


def _head_body(pu_ref, pv_ref, pwu_ref, pwv_ref, pb_ref, plg_ref, plb_ref,
               qw_ref, qb_ref, gvec_ref, ghw_ref, ghb_ref, ghg_ref, ghbb_ref,
               qgw_ref, qgb_ref, mask_ref, out_ref):
    pre = (
        jnp.dot(pu_ref[:], pwu_ref[:], preferred_element_type=jnp.float32)
        + jnp.dot(pv_ref[:], pwv_ref[:], preferred_element_type=jnp.float32)
        + pb_ref[:]
    )
    r = jnp.maximum(pre, 0.0)
    mu = jnp.mean(r, axis=1, keepdims=True)
    var = jnp.mean((r - mu) ** 2, axis=1, keepdims=True)
    npf = (r - mu) * lax.rsqrt(var + 1e-5) * plg_ref[:] + plb_ref[:]
    qn = jnp.dot(npf, qw_ref[:], preferred_element_type=jnp.float32) + qb_ref[:]

    g = gvec_ref[:]
    gr = jnp.maximum(
        jnp.dot(g, ghw_ref[:], preferred_element_type=jnp.float32) + ghb_ref[:],
        0.0,
    )
    gmu = jnp.mean(gr, axis=1, keepdims=True)
    gvar = jnp.mean((gr - gmu) ** 2, axis=1, keepdims=True)
    gh = (gr - gmu) * lax.rsqrt(gvar + 1e-5) * ghg_ref[:] + ghbb_ref[:]
    qg = jnp.dot(gh, qgw_ref[:], preferred_element_type=jnp.float32) + qgb_ref[:]

    q = qn + qg
    m = mask_ref[:]
    neg = jnp.float32(-1e9)
    o0 = jnp.where(m < 0.5, neg, q[:, 0:1])
    o1 = jnp.where((1.0 - m) < 0.5, neg, q[:, 1:2])
    out_ref[:] = jnp.concatenate([o0, o1], axis=1)


_T_HEAD = 2048


def _tc_head(pu, pv, pwu, pwv, pbr, plgr, plbr, qw, qbr,
             gvec, ghw, ghbr, ghgr, ghbbr, qgw, qgbr, maskc):
    row = lambda w: pl.BlockSpec((_T_HEAD, w), lambda i: (i, 0))
    full = lambda s: pl.BlockSpec(s, lambda i: tuple(0 for _ in s))
    return pl.pallas_call(
        _head_body,
        grid=(_PP // _T_HEAD,),
        in_specs=[
            row(32), row(32),
            full((32, 32)), full((32, 32)), full((1, 32)),
            full((1, 32)), full((1, 32)),
            full((32, 2)), full((1, 2)),
            full((1, 32)), full((32, 32)), full((1, 32)),
            full((1, 32)), full((1, 32)),
            full((32, 2)), full((1, 2)),
            row(1),
        ],
        out_specs=row(2),
        out_shape=jax.ShapeDtypeStruct((_PP, 2), jnp.float32),
    )(pu, pv, pwu, pwv, pbr, plgr, plbr, qw, qbr,
      gvec, ghw, ghbr, ghgr, ghbbr, qgw, qgbr, maskc)


def kernel(x, edge_index, edge_attr, pipe_edge_idx, pipe_open_mask, batch, params):
    p = params
    f32 = jnp.float32
    i32 = jnp.int32

    src = edge_index[0]
    dst = edge_index[1]
    epad = _EP - _E
    srcp = jnp.concatenate([src, jnp.zeros((epad,), i32)]).reshape(_EP // 128, 128)
    dstp = jnp.concatenate([dst, jnp.full((epad,), _N, i32)]).reshape(_EP // 128, 128)
    xpad = jnp.pad(x, ((0, _NP - _N), (0, 0))).astype(jnp.bfloat16)
    zeros_tab = jnp.zeros((_NP, _ACC_W), f32)
    eit = jnp.pad(edge_index.T, ((0, 0), (0, 6)))
    pipe2d = jnp.pad(pipe_edge_idx, (0, _PP - _P)).reshape(_PP // 128, 128)
    maskc = jnp.pad(pipe_open_mask, (0, _PP - _P)).reshape(_PP, 1)

    bf16 = jnp.bfloat16
    w1a, b1a = p["enn1_w1"], p["enn1_b1"].reshape(1, -1)
    w2a, b2a = p["enn1_w2"].astype(bf16), p["enn1_b2"].reshape(1, -1)
    w1b, b1b = p["enn2_w1"], p["enn2_b1"].reshape(1, -1)
    w2b, b2b = p["enn2_w2"].astype(bf16), p["enn2_b2"].reshape(1, -1)

    def rep_mat(nin):
        nw = nin * 32
        return (jnp.arange(nw)[None, :] // 32
                == jnp.arange(nin)[:, None]).astype(bf16)

    rep64 = rep_mat(64)
    rep32 = rep_mat(32)

    xj = _sc_gather_rows(xpad, srcp, _EP)
    msg1 = _tc_msg(edge_attr, xj, w1a, b1a, w2a, b2a, rep64, 64)
    parts1 = [_sc_segment_acc(msg1, dstp, zeros_tab)]
    h1 = _tc_node(parts1, xpad, p["root1"].astype(bf16), p["bias1"].reshape(1, -1),
                  p["ln1_g"].reshape(1, -1), p["ln1_b"].reshape(1, -1))

    hj = _sc_gather_rows(h1, srcp, _EP)
    msg2 = _tc_msg(edge_attr, hj, w1b, b1b, w2b, b2b, rep32, 32)
    parts2 = [_sc_segment_acc(msg2, dstp, zeros_tab)]
    node_emb, gvec = _tc_node2(parts2, h1, p["root2"].astype(bf16), p["bias2"].reshape(1, -1),
                               p["ln2_g"].reshape(1, -1), p["ln2_b"].reshape(1, -1))

    pu, pv = _sc_pair_gather(eit, pipe2d, node_emb)
    qs = _tc_head(
        pu, pv,
        p["pair_w"][:32].astype(bf16), p["pair_w"][32:].astype(bf16),
        p["pair_b"].reshape(1, -1),
        p["pair_ln_g"].reshape(1, -1), p["pair_ln_b"].reshape(1, -1),
        p["qnode_w"], p["qnode_b"].reshape(1, -1),
        gvec, p["gh_w"], p["gh_b"].reshape(1, -1),
        p["gh_ln_g"].reshape(1, -1), p["gh_ln_b"].reshape(1, -1),
        p["qg_w"], p["qg_b"].reshape(1, -1),
        maskc,
    )
    return qs[:_P].reshape(1, -1)
